# baseline (speedup 1.0000x reference)
import functools

import jax
import jax.numpy as jnp
from jax import lax
from jax.experimental import pallas as pl
from jax.experimental.pallas import tpu as pltpu
from jax.experimental.pallas import tpu_sc as plsc

N = 160000
D = 256
NUM_SEG = 10000

ROWS_PER_BLOCK = 8192
NUM_BLOCKS = -(-N // ROWS_PER_BLOCK)

NUM_TILES = 16
ROWS_PER_TILE = N // NUM_TILES
VECS_PER_TILE = ROWS_PER_TILE // 16
SEG_PAD = 10240
CHUNK = SEG_PAD // NUM_TILES
CHUNK_VECS = CHUNK // 16


def _matvec_body(x1_ref, x2_ref, wt_ref, o1_ref, o2_ref):
    o1_ref[...] = jnp.sum(x1_ref[...] * wt_ref[...], axis=1).reshape(
        ROWS_PER_BLOCK // 128, 128)
    o2_ref[...] = jnp.sum(x2_ref[...] * wt_ref[...], axis=1).reshape(
        ROWS_PER_BLOCK // 128, 128)


def _matvec(x, W):
    wt = W.reshape(1, D)
    half_blocks = NUM_BLOCKS // 2
    split = half_blocks * ROWS_PER_BLOCK
    o1, o2 = pl.pallas_call(
        _matvec_body,
        grid=(half_blocks,),
        in_specs=[
            pl.BlockSpec((ROWS_PER_BLOCK, D), lambda i: (i, 0)),
            pl.BlockSpec((ROWS_PER_BLOCK, D),
                         lambda i: (i + half_blocks, 0)),
            pl.BlockSpec((1, D), lambda i: (0, 0)),
        ],
        out_specs=[
            pl.BlockSpec((ROWS_PER_BLOCK // 128, 128), lambda i: (i, 0)),
            pl.BlockSpec((ROWS_PER_BLOCK // 128, 128), lambda i: (i, 0)),
        ],
        out_shape=[
            jax.ShapeDtypeStruct((split // 128, 128), jnp.float32),
            jax.ShapeDtypeStruct((split // 128, 128), jnp.float32),
        ],
    )(x, x, wt)
    v = jnp.concatenate([o1.reshape(split), o2.reshape(split)])
    return v[:N]


def _segsum_body(v_hbm, ids_hbm, bvec_hbm, out_hbm,
                 vals_v, ids_v, acc_v, red_v, out_v, bvec_v, shared,
                 sem1, sem2, sem3):
    c = lax.axis_index("c")
    s = lax.axis_index("s")

    h1 = pltpu.async_copy(
        v_hbm.at[pl.ds(s * ROWS_PER_TILE, ROWS_PER_TILE)], vals_v, sem1)
    h2 = pltpu.async_copy(
        ids_hbm.at[pl.ds(s * ROWS_PER_TILE, ROWS_PER_TILE)], ids_v, sem2)
    h3 = pltpu.async_copy(bvec_hbm, bvec_v, sem3)

    def zero_body(j, _):
        acc_v[pl.ds(j * 16, 16)] = jnp.zeros((16,), jnp.float32)
        return 0
    lax.fori_loop(0, SEG_PAD // 16, zero_body, 0)
    h1.wait()
    h2.wait()

    lanes = jnp.arange(16, dtype=jnp.int32) * VECS_PER_TILE

    def scat_body(j, _):
        for u in range(25):
            pos = lanes + (25 * j + u)
            idv = plsc.load_gather(ids_v, [pos])
            vv = plsc.load_gather(vals_v, [pos])
            plsc.addupdate_scatter(acc_v, [idv], vv)
        return 0
    lax.fori_loop(0, VECS_PER_TILE // 25, scat_body, 0)

    pltpu.sync_copy(acc_v, shared.at[s])
    plsc.subcore_barrier()
    pltpu.sync_copy(shared.at[:, pl.ds(s * CHUNK, CHUNK)], red_v)
    h3.wait()

    def merge_body(j, _):
        tot = bvec_v[...]
        for t in range(NUM_TILES):
            tot = tot + red_v[t, pl.ds(j * 16, 16)]
        out_v[pl.ds(j * 16, 16)] = tot
        return 0
    lax.fori_loop(0, CHUNK_VECS, merge_body, 0)

    half = NUM_TILES // 2

    @pl.when(jnp.logical_and(c == 0, s < half))
    def _():
        pltpu.sync_copy(out_v, out_hbm.at[pl.ds(s * CHUNK, CHUNK)])

    @pl.when(jnp.logical_and(c == 1, jnp.logical_and(s >= half, s < NUM_TILES - 1)))
    def _():
        pltpu.sync_copy(out_v, out_hbm.at[pl.ds(s * CHUNK, CHUNK)])

    last = NUM_SEG - (NUM_TILES - 1) * CHUNK

    @pl.when(jnp.logical_and(c == 1, s == NUM_TILES - 1))
    def _():
        pltpu.sync_copy(out_v.at[pl.ds(0, last)],
                        out_hbm.at[pl.ds((NUM_TILES - 1) * CHUNK, last)])


def _segsum(v, ids, bvec):
    mesh = plsc.VectorSubcoreMesh(core_axis_name="c", subcore_axis_name="s")
    f = functools.partial(
        pl.kernel,
        out_type=jax.ShapeDtypeStruct((NUM_SEG,), jnp.float32),
        mesh=mesh,
        scratch_types=[
            pltpu.VMEM((ROWS_PER_TILE,), jnp.float32),
            pltpu.VMEM((ROWS_PER_TILE,), jnp.int32),
            pltpu.VMEM((SEG_PAD,), jnp.float32),
            pltpu.VMEM((NUM_TILES, CHUNK), jnp.float32),
            pltpu.VMEM((CHUNK,), jnp.float32),
            pltpu.VMEM((16,), jnp.float32),
            pltpu.VMEM_SHARED((NUM_TILES, SEG_PAD), jnp.float32),
            pltpu.SemaphoreType.DMA,
            pltpu.SemaphoreType.DMA,
            pltpu.SemaphoreType.DMA,
        ],
        compiler_params=pltpu.CompilerParams(needs_layout_passes=False),
    )(_segsum_body)
    return f(v, ids, bvec)


@jax.jit
def kernel(x, y, z, W, b):
    del z
    v = _matvec(x, W)
    ids = y.reshape(N).astype(jnp.int32)
    bvec = jnp.broadcast_to(b.astype(jnp.float32), (16,))
    out = _segsum(v, ids, bvec)
    return out.reshape(NUM_SEG, 1)


# speedup vs baseline: 1.0551x; 1.0551x over previous
import functools

import jax
import jax.numpy as jnp
from jax import lax
from jax.experimental import pallas as pl
from jax.experimental.pallas import tpu as pltpu
from jax.experimental.pallas import tpu_sc as plsc

N = 160000
D = 256
NUM_SEG = 10000

ROWS_PER_BLOCK = 8192
NUM_BLOCKS = -(-N // ROWS_PER_BLOCK)

NUM_TILES = 16
ROWS_PER_TILE = N // NUM_TILES
VECS_PER_TILE = ROWS_PER_TILE // 16
SEG_PAD = 10240
CHUNK = SEG_PAD // NUM_TILES
CHUNK_VECS = CHUNK // 16


def _matvec_body(x_ref, wt_ref, o_ref):
    o_ref[...] = jnp.sum(x_ref[...] * wt_ref[...], axis=1).reshape(
        ROWS_PER_BLOCK // 128, 128)


def _matvec(x, W):
    wt = W.reshape(1, D)
    v2 = pl.pallas_call(
        _matvec_body,
        grid=(NUM_BLOCKS,),
        in_specs=[
            pl.BlockSpec((ROWS_PER_BLOCK, D), lambda i: (i, 0)),
            pl.BlockSpec((1, D), lambda i: (0, 0)),
        ],
        out_specs=pl.BlockSpec((ROWS_PER_BLOCK // 128, 128),
                               lambda i: (i, 0)),
        out_shape=jax.ShapeDtypeStruct((N // 128, 128), jnp.float32),
    )(x, wt)
    return v2.reshape(N)


def _segsum_body(v_hbm, ids_hbm, bvec_hbm, out_hbm,
                 vals_v, ids_v, acc_v, red_v, out_v, bvec_v, shared,
                 sem1, sem2, sem3):
    c = lax.axis_index("c")
    s = lax.axis_index("s")

    h1 = pltpu.async_copy(
        v_hbm.at[pl.ds(s * ROWS_PER_TILE, ROWS_PER_TILE)], vals_v, sem1)
    h2 = pltpu.async_copy(
        ids_hbm.at[pl.ds(s * ROWS_PER_TILE, ROWS_PER_TILE)], ids_v, sem2)
    h3 = pltpu.async_copy(bvec_hbm, bvec_v, sem3)

    def zero_body(j, _):
        acc_v[pl.ds(j * 16, 16)] = jnp.zeros((16,), jnp.float32)
        return 0
    lax.fori_loop(0, SEG_PAD // 16, zero_body, 0)
    h1.wait()
    h2.wait()

    lanes = jnp.arange(16, dtype=jnp.int32) * VECS_PER_TILE

    def scat_body(j, _):
        for u in range(25):
            pos = lanes + (25 * j + u)
            idv = plsc.load_gather(ids_v, [pos])
            vv = plsc.load_gather(vals_v, [pos])
            plsc.addupdate_scatter(acc_v, [idv], vv)
        return 0
    lax.fori_loop(0, VECS_PER_TILE // 25, scat_body, 0)

    pltpu.sync_copy(acc_v, shared.at[s])
    plsc.subcore_barrier()
    pltpu.sync_copy(shared.at[:, pl.ds(s * CHUNK, CHUNK)], red_v)
    h3.wait()

    def merge_body(j, _):
        tot = bvec_v[...]
        for t in range(NUM_TILES):
            tot = tot + red_v[t, pl.ds(j * 16, 16)]
        out_v[pl.ds(j * 16, 16)] = tot
        return 0
    lax.fori_loop(0, CHUNK_VECS, merge_body, 0)

    half = NUM_TILES // 2

    @pl.when(jnp.logical_and(c == 0, s < half))
    def _():
        pltpu.sync_copy(out_v, out_hbm.at[pl.ds(s * CHUNK, CHUNK)])

    @pl.when(jnp.logical_and(c == 1, jnp.logical_and(s >= half, s < NUM_TILES - 1)))
    def _():
        pltpu.sync_copy(out_v, out_hbm.at[pl.ds(s * CHUNK, CHUNK)])

    last = NUM_SEG - (NUM_TILES - 1) * CHUNK

    @pl.when(jnp.logical_and(c == 1, s == NUM_TILES - 1))
    def _():
        pltpu.sync_copy(out_v.at[pl.ds(0, last)],
                        out_hbm.at[pl.ds((NUM_TILES - 1) * CHUNK, last)])


def _segsum(v, ids, bvec):
    mesh = plsc.VectorSubcoreMesh(core_axis_name="c", subcore_axis_name="s")
    f = functools.partial(
        pl.kernel,
        out_type=jax.ShapeDtypeStruct((NUM_SEG,), jnp.float32),
        mesh=mesh,
        scratch_types=[
            pltpu.VMEM((ROWS_PER_TILE,), jnp.float32),
            pltpu.VMEM((ROWS_PER_TILE,), jnp.int32),
            pltpu.VMEM((SEG_PAD,), jnp.float32),
            pltpu.VMEM((NUM_TILES, CHUNK), jnp.float32),
            pltpu.VMEM((CHUNK,), jnp.float32),
            pltpu.VMEM((16,), jnp.float32),
            pltpu.VMEM_SHARED((NUM_TILES, SEG_PAD), jnp.float32),
            pltpu.SemaphoreType.DMA,
            pltpu.SemaphoreType.DMA,
            pltpu.SemaphoreType.DMA,
        ],
        compiler_params=pltpu.CompilerParams(needs_layout_passes=False),
    )(_segsum_body)
    return f(v, ids, bvec)


@jax.jit
def kernel(x, y, z, W, b):
    del z
    v = _matvec(x, W)
    ids = y.reshape(N).astype(jnp.int32)
    bvec = jnp.broadcast_to(b.astype(jnp.float32), (16,))
    out = _segsum(v, ids, bvec)
    return out.reshape(NUM_SEG, 1)
